# baseline (speedup 1.0000x reference)
import jax
import jax.numpy as jnp
from jax.experimental import pallas as pl
from jax.experimental.pallas import tpu as pltpu

_M = 1024
_B = 512
_TS = (0.3 / 3600.0) ** 0.5


_C = 8


def _tile_kernel(dd_ref, sdt_ref, cd_ref, sct_ref, hlp_ref, asn_ref, w_ref):
    dd = jnp.broadcast_to(dd_ref[:], (_C, _B))
    sdt = jnp.broadcast_to(sdt_ref[:], (_C, _B))

    def body(c, carry):
        rmin, ridx = carry
        base = c * _C
        cd_c = cd_ref[pl.ds(base, _C), :]
        sct_c = sct_ref[pl.ds(base, _C), :]
        hlp_c = hlp_ref[pl.ds(base, _C), :]
        d1 = dd - cd_c
        t1 = sdt - sct_c
        cost_c = (d1 * d1 + hlp_c) + t1 * t1
        pred = cost_c < rmin
        rmin = jnp.where(pred, cost_c, rmin)
        ridx = jnp.where(pred, c, ridx)
        return rmin, ridx

    rmin0 = jnp.full((_C, _B), jnp.inf, jnp.float32)
    ridx0 = jnp.zeros((_C, _B), jnp.int32)
    rmin, ridx = jax.lax.fori_loop(0, _M // _C, body, (rmin0, ridx0),
                                   unroll=4)

    j8 = ridx * _C + jax.lax.broadcasted_iota(jnp.int32, (_C, _B), 0)
    min_cost = jnp.min(rmin, axis=0, keepdims=True)
    min_j = jnp.min(jnp.where(rmin == min_cost, j8, _M), axis=0,
                    keepdims=True)

    valid = min_cost < 625.0
    asn_ref[:] = jnp.where(valid, min_j, -1)
    w_ref[:] = jnp.where(valid, 1.0 / (1.0 + jnp.sqrt(min_cost)), 0.0)


def kernel(detection_depths, camera_depths, detection_times, camera_times):
    n = detection_depths.shape[0]
    m = camera_depths.shape[0]
    dd = detection_depths.reshape(1, n)
    sdt = (detection_times * _TS).reshape(1, n)
    cd = camera_depths.reshape(m, 1)
    sct = (camera_times * _TS).reshape(m, 1)
    hlp = (0.5 * (1.0 - jnp.exp(-0.045 * camera_depths))).reshape(m, 1)

    grid = (n // _B,)
    asn, w = pl.pallas_call(
        _tile_kernel,
        grid=grid,
        in_specs=[
            pl.BlockSpec((1, _B), lambda i: (0, i)),
            pl.BlockSpec((1, _B), lambda i: (0, i)),
            pl.BlockSpec((m, 1), lambda i: (0, 0)),
            pl.BlockSpec((m, 1), lambda i: (0, 0)),
            pl.BlockSpec((m, 1), lambda i: (0, 0)),
        ],
        out_specs=[
            pl.BlockSpec((1, _B), lambda i: (0, i)),
            pl.BlockSpec((1, _B), lambda i: (0, i)),
        ],
        out_shape=[
            jax.ShapeDtypeStruct((1, n), jnp.int32),
            jax.ShapeDtypeStruct((1, n), jnp.float32),
        ],
        compiler_params=pltpu.CompilerParams(
            dimension_semantics=("parallel",)),
    )(dd, sdt, cd, sct, hlp)

    assignments = asn.reshape(n).astype(jnp.int64)
    weights = w.reshape(n)
    return assignments, weights


# speedup vs baseline: 2.8551x; 2.8551x over previous
import jax
import jax.numpy as jnp
from jax.experimental import pallas as pl
from jax.experimental.pallas import tpu as pltpu

_M = 1024
_B = 512
_TS = (0.3 / 3600.0) ** 0.5


def _tile_kernel(dd_ref, sdt_ref, cd_ref, sct_ref, hlp_ref, idsf_ref,
                 asn_ref, w_ref):
    dd = dd_ref[:]
    sdt = sdt_ref[:]
    cd = cd_ref[:]
    sct = sct_ref[:]
    hlp = hlp_ref[:]
    idsf = idsf_ref[:]

    d1 = dd - cd
    t1 = sdt - sct
    cost = (d1 * d1 + hlp) + t1 * t1

    min_cost = jnp.min(cost, axis=0, keepdims=True)
    min_jf = jnp.min(jnp.where(cost == min_cost, idsf, 2048.0), axis=0,
                     keepdims=True)
    min_j = min_jf.astype(jnp.int32)

    valid = min_cost < 625.0
    asn_ref[:] = jnp.where(valid, min_j, -1)
    w_ref[:] = jnp.where(valid, 1.0 / (1.0 + jnp.sqrt(min_cost)), 0.0)


def kernel(detection_depths, camera_depths, detection_times, camera_times):
    n = detection_depths.shape[0]
    m = camera_depths.shape[0]
    dd = detection_depths.reshape(1, n)
    sdt = (detection_times * _TS).reshape(1, n)
    cd = camera_depths.reshape(m, 1)
    sct = (camera_times * _TS).reshape(m, 1)
    hlp = (0.5 * (1.0 - jnp.exp(-0.045 * camera_depths))).reshape(m, 1)
    idsf = jnp.arange(m, dtype=jnp.float32).reshape(m, 1)

    grid = (n // _B,)
    asn, w = pl.pallas_call(
        _tile_kernel,
        grid=grid,
        in_specs=[
            pl.BlockSpec((1, _B), lambda i: (0, i)),
            pl.BlockSpec((1, _B), lambda i: (0, i)),
            pl.BlockSpec((m, 1), lambda i: (0, 0)),
            pl.BlockSpec((m, 1), lambda i: (0, 0)),
            pl.BlockSpec((m, 1), lambda i: (0, 0)),
            pl.BlockSpec((m, 1), lambda i: (0, 0)),
        ],
        out_specs=[
            pl.BlockSpec((1, _B), lambda i: (0, i)),
            pl.BlockSpec((1, _B), lambda i: (0, i)),
        ],
        out_shape=[
            jax.ShapeDtypeStruct((1, n), jnp.int32),
            jax.ShapeDtypeStruct((1, n), jnp.float32),
        ],
        compiler_params=pltpu.CompilerParams(
            dimension_semantics=("parallel",)),
    )(dd, sdt, cd, sct, hlp, idsf)

    assignments = asn.reshape(n).astype(jnp.int64)
    weights = w.reshape(n)
    return assignments, weights
